# baseline (speedup 1.0000x reference)
import functools

import jax
import jax.numpy as jnp
import numpy as np
from jax import lax
from jax.experimental import pallas as pl
from jax.experimental.pallas import tpu as pltpu
from jax.experimental.pallas import tpu_sc as plsc

_THR = np.float32(0.3)
_BIG_I32 = np.int32(2**30)
_NEG_INF = np.float32(-np.inf)

_SC_CORES = 2
_SC_SUBCORES = 16
_SC_WORKERS = _SC_CORES * _SC_SUBCORES
_SC_LANES = 16


def _sim_body(dt_ref, dt1_ref, w1_ref, b1_ref, w2_ref, b2_ref,
              sim_ref, rmax_ref, rarg_ref, a3_s, b3_s):
    step = pl.program_id(0)
    B = sim_ref.shape[0]
    M = sim_ref.shape[1]

    @pl.when(step == 0)
    def _init():
        w1 = w1_ref[...]
        b1 = b1_ref[...]
        w2 = w2_ref[...]
        b2 = b2_ref[...]

        def emb(x):
            h = jnp.maximum(
                lax.dot_general(x, w1, (((1,), (0,)), ((), ())),
                                preferred_element_type=jnp.float32) + b1, 0.0)
            y = lax.dot_general(h, w2, (((1,), (0,)), ((), ())),
                                preferred_element_type=jnp.float32) + b2
            n = jnp.sqrt(jnp.sum(y * y, axis=1, keepdims=True))
            return y / jnp.maximum(n, 1e-12)

        e = emb(dt_ref[...])
        hi = e.astype(jnp.bfloat16)
        lo = (e - hi.astype(jnp.float32)).astype(jnp.bfloat16)
        a3_s[...] = jnp.concatenate([hi, hi, lo], axis=1)
        f = emb(dt1_ref[...])
        fhi = f.astype(jnp.bfloat16)
        flo = (f - fhi.astype(jnp.float32)).astype(jnp.bfloat16)
        b3_s[...] = jnp.concatenate([fhi, flo, fhi], axis=1)

    a_blk = a3_s[pl.ds(step * B, B), :]
    sim = lax.dot_general(a_blk, b3_s[...], (((1,), (1,)), ((), ())),
                          preferred_element_type=jnp.float32)
    sim_ref[...] = sim
    m = jnp.max(sim, axis=1, keepdims=True)
    iota_j = lax.broadcasted_iota(jnp.int32, (B, M), 1)
    ji = jnp.min(jnp.where(sim == m, iota_j, M), axis=1, keepdims=True)
    rmax_ref[...] = m
    rarg_ref[...] = ji


def _scatter_body(chunk, n_pad, n_valid,
                  ms_hbm, ji_hbm, cwv0_hbm, cwi0_hbm, cwv1_hbm, cwi1_hbm,
                  ms_v, ji_v, bestv, besti, shv, shi, tv, ti, mgv, mgi):
    core = lax.axis_index("c")
    sid = lax.axis_index("s")
    wid = sid * _SC_CORES + core
    base = wid * chunk
    L = _SC_LANES
    stripe = n_pad // _SC_SUBCORES

    pltpu.sync_copy(ms_hbm.at[pl.ds(base, chunk)], ms_v)
    pltpu.sync_copy(ji_hbm.at[pl.ds(base, chunk)], ji_v)

    ninf = jnp.full((L,), _NEG_INF, jnp.float32)
    big = jnp.full((L,), _BIG_I32, jnp.int32)
    for g in range(n_pad // L):
        bestv[pl.ds(g * L, L)] = ninf
        besti[pl.ds(g * L, L)] = big

    lane_iota = lax.iota(jnp.int32, L)
    for r in range(chunk):
        g, lane = divmod(r, L)
        jv = ji_v[pl.ds(g * L, L)]
        jv = jnp.minimum(jnp.maximum(jv, 0), np.int32(n_pad - 1))
        mv = ms_v[pl.ds(g * L, L)]
        iv = lane_iota + (base + g * L)
        gv = plsc.load_gather(bestv, [jv])
        gi = plsc.load_gather(besti, [jv])
        win = (mv > gv) | ((mv == gv) & (iv < gi))
        msk = (lane_iota == lane) & win & (iv < n_valid)
        plsc.store_scatter(bestv, [jv], mv, mask=msk)
        plsc.store_scatter(besti, [jv], iv, mask=msk)

    pltpu.sync_copy(bestv, shv.at[pl.ds(sid * n_pad, n_pad)])
    pltpu.sync_copy(besti, shi.at[pl.ds(sid * n_pad, n_pad)])
    plsc.subcore_barrier()
    pltpu.sync_copy(shv.at[pl.ds(sid * stripe, stripe)], tv)
    pltpu.sync_copy(shi.at[pl.ds(sid * stripe, stripe)], ti)
    for g in range(stripe // L):
        mgv[pl.ds(g * L, L)] = tv[pl.ds(g * L, L)]
        mgi[pl.ds(g * L, L)] = ti[pl.ds(g * L, L)]
    for s2 in range(1, _SC_SUBCORES):
        pltpu.sync_copy(shv.at[pl.ds(s2 * n_pad + sid * stripe, stripe)], tv)
        pltpu.sync_copy(shi.at[pl.ds(s2 * n_pad + sid * stripe, stripe)], ti)
        for g in range(stripe // L):
            av = mgv[pl.ds(g * L, L)]
            ai = mgi[pl.ds(g * L, L)]
            bv = tv[pl.ds(g * L, L)]
            bi = ti[pl.ds(g * L, L)]
            better = (bv > av) | ((bv == av) & (bi < ai))
            mgv[pl.ds(g * L, L)] = jnp.where(better, bv, av)
            mgi[pl.ds(g * L, L)] = jnp.where(better, bi, ai)

    @pl.when(core == 0)
    def _w0():
        pltpu.sync_copy(mgv, cwv0_hbm.at[pl.ds(sid * stripe, stripe)])
        pltpu.sync_copy(mgi, cwi0_hbm.at[pl.ds(sid * stripe, stripe)])

    @pl.when(core == 1)
    def _w1():
        pltpu.sync_copy(mgv, cwv1_hbm.at[pl.ds(sid * stripe, stripe)])
        pltpu.sync_copy(mgi, cwi1_hbm.at[pl.ds(sid * stripe, stripe)])


def _resolve_body(chunk, n_pad,
                  ms_hbm, ji_hbm, cwv0_hbm, cwi0_hbm, cwv1_hbm, cwi1_hbm,
                  out_hbm, ms_v, ji_v, v0, i0, v1, i1, out_v):
    wid = lax.axis_index("s") * _SC_CORES + lax.axis_index("c")
    base = wid * chunk
    L = _SC_LANES
    pltpu.sync_copy(ms_hbm.at[pl.ds(base, chunk)], ms_v)
    pltpu.sync_copy(ji_hbm.at[pl.ds(base, chunk)], ji_v)
    pltpu.sync_copy(cwv0_hbm, v0)
    pltpu.sync_copy(cwi0_hbm, i0)
    pltpu.sync_copy(cwv1_hbm, v1)
    pltpu.sync_copy(cwi1_hbm, i1)
    for k in range(chunk // L):
        off = k * L
        jraw = ji_v[pl.ds(off, L)]
        jv = jnp.minimum(jnp.maximum(jraw, 0), np.int32(n_pad - 1))
        mv = ms_v[pl.ds(off, L)]
        gv0 = plsc.load_gather(v0, [jv])
        gi0 = plsc.load_gather(i0, [jv])
        gv1 = plsc.load_gather(v1, [jv])
        gi1 = plsc.load_gather(i1, [jv])
        first = (gv0 > gv1) | ((gv0 == gv1) & (gi0 < gi1))
        w = jnp.where(first, gi0, gi1)
        iv = lax.iota(jnp.int32, L) + (base + off)
        out_v[pl.ds(off, L)] = jnp.where(
            (mv > _THR) & (w == iv), jraw, jnp.int32(-1))
    pltpu.sync_copy(out_v, out_hbm.at[pl.ds(base, chunk)])


def kernel(detections_t, detections_t1, W1, b1, W2, b2):
    N_t, D = detections_t.shape
    N_t1 = detections_t1.shape[0]
    E = W2.shape[1]

    chunk = -(-N_t // (_SC_WORKERS * _SC_LANES)) * _SC_LANES
    n_pad = chunk * _SC_WORKERS

    BLK = 200
    grid = N_t // BLK
    sim, rmax, rarg = pl.pallas_call(
        _sim_body,
        grid=(grid,),
        in_specs=[
            pl.BlockSpec((N_t, D), lambda i: (0, 0)),
            pl.BlockSpec((N_t1, D), lambda i: (0, 0)),
            pl.BlockSpec((D, E), lambda i: (0, 0)),
            pl.BlockSpec((1, E), lambda i: (0, 0)),
            pl.BlockSpec((E, E), lambda i: (0, 0)),
            pl.BlockSpec((1, E), lambda i: (0, 0)),
        ],
        out_specs=[
            pl.BlockSpec((BLK, N_t1), lambda i: (i, 0)),
            pl.BlockSpec((BLK, 1), lambda i: (i, 0)),
            pl.BlockSpec((BLK, 1), lambda i: (i, 0)),
        ],
        out_shape=(
            jax.ShapeDtypeStruct((N_t, N_t1), jnp.float32),
            jax.ShapeDtypeStruct((n_pad, 1), jnp.float32),
            jax.ShapeDtypeStruct((n_pad, 1), jnp.int32),
        ),
        scratch_shapes=[
            pltpu.VMEM((N_t, 3 * E), jnp.bfloat16),
            pltpu.VMEM((N_t1, 3 * E), jnp.bfloat16),
        ],
    )(detections_t, detections_t1, W1, b1.reshape(1, E), W2,
      b2.reshape(1, E))

    mesh = plsc.VectorSubcoreMesh(core_axis_name="c", subcore_axis_name="s",
                                  num_cores=_SC_CORES,
                                  num_subcores=_SC_SUBCORES)
    stripe = n_pad // _SC_SUBCORES
    scatter = functools.partial(
        pl.kernel,
        out_type=(
            jax.ShapeDtypeStruct((n_pad,), jnp.float32),
            jax.ShapeDtypeStruct((n_pad,), jnp.int32),
            jax.ShapeDtypeStruct((n_pad,), jnp.float32),
            jax.ShapeDtypeStruct((n_pad,), jnp.int32),
        ),
        mesh=mesh,
        compiler_params=pltpu.CompilerParams(needs_layout_passes=False),
        scratch_types=[
            pltpu.VMEM((chunk,), jnp.float32),
            pltpu.VMEM((chunk,), jnp.int32),
            pltpu.VMEM((n_pad,), jnp.float32),
            pltpu.VMEM((n_pad,), jnp.int32),
            pltpu.VMEM_SHARED((_SC_SUBCORES * n_pad,), jnp.float32),
            pltpu.VMEM_SHARED((_SC_SUBCORES * n_pad,), jnp.int32),
            pltpu.VMEM((stripe,), jnp.float32),
            pltpu.VMEM((stripe,), jnp.int32),
            pltpu.VMEM((stripe,), jnp.float32),
            pltpu.VMEM((stripe,), jnp.int32),
        ],
    )(functools.partial(_scatter_body, chunk, n_pad, N_t))
    ms_flat = rmax.reshape(n_pad)
    ji_flat = rarg.reshape(n_pad)
    cwv0, cwi0, cwv1, cwi1 = scatter(ms_flat, ji_flat)

    resolve = functools.partial(
        pl.kernel,
        out_type=jax.ShapeDtypeStruct((n_pad,), jnp.int32),
        mesh=mesh,
        compiler_params=pltpu.CompilerParams(needs_layout_passes=False),
        scratch_types=[
            pltpu.VMEM((chunk,), jnp.float32),
            pltpu.VMEM((chunk,), jnp.int32),
            pltpu.VMEM((n_pad,), jnp.float32),
            pltpu.VMEM((n_pad,), jnp.int32),
            pltpu.VMEM((n_pad,), jnp.float32),
            pltpu.VMEM((n_pad,), jnp.int32),
            pltpu.VMEM((chunk,), jnp.int32),
        ],
    )(functools.partial(_resolve_body, chunk, n_pad))
    matches = resolve(ms_flat, ji_flat, cwv0, cwi0, cwv1, cwi1)[:N_t]
    return matches, sim


# speedup vs baseline: 1.1479x; 1.1479x over previous
import functools

import jax
import jax.numpy as jnp
import numpy as np
from jax import lax
from jax.experimental import pallas as pl
from jax.experimental.pallas import tpu as pltpu
from jax.experimental.pallas import tpu_sc as plsc

_THR = np.float32(0.3)
_BIG_I32 = np.int32(2**30)
_NEG_INF = np.float32(-np.inf)

_SC_CORES = 2
_SC_SUBCORES = 16
_SC_WORKERS = _SC_CORES * _SC_SUBCORES
_SC_LANES = 16


def _sim_body(dt_ref, dt1_ref, w1_ref, b1_ref, w2_ref, b2_ref,
              sim_ref, rmax_ref, rarg_ref, et_s, et1_s):
    step = pl.program_id(0)
    B = sim_ref.shape[0]
    M = sim_ref.shape[1]

    @pl.when(step == 0)
    def _init():
        w1 = w1_ref[...]
        b1 = b1_ref[...]
        w2 = w2_ref[...]
        b2 = b2_ref[...]

        def emb(x):
            h = jnp.maximum(
                lax.dot_general(x, w1, (((1,), (0,)), ((), ())),
                                preferred_element_type=jnp.float32) + b1, 0.0)
            y = lax.dot_general(h, w2, (((1,), (0,)), ((), ())),
                                preferred_element_type=jnp.float32) + b2
            n = jnp.sqrt(jnp.sum(y * y, axis=1, keepdims=True))
            return y / jnp.maximum(n, 1e-12)

        et_s[...] = emb(dt_ref[...])
        et1_s[...] = emb(dt1_ref[...])

    e_blk = et_s[pl.ds(step * B, B), :]
    sim = lax.dot_general(e_blk, et1_s[...], (((1,), (1,)), ((), ())),
                          preferred_element_type=jnp.float32)
    sim_ref[...] = sim
    m = jnp.max(sim, axis=1, keepdims=True)
    iota_j = lax.broadcasted_iota(jnp.int32, (B, M), 1)
    ji = jnp.min(jnp.where(sim == m, iota_j, M), axis=1, keepdims=True)
    rmax_ref[...] = m
    rarg_ref[...] = ji


def _scatter_body(chunk, n_pad, n_valid,
                  ms_hbm, ji_hbm, cwv0_hbm, cwi0_hbm, cwv1_hbm, cwi1_hbm,
                  ms_v, ji_v, bestv, besti, shv, shi, tv, ti, mgv, mgi):
    core = lax.axis_index("c")
    sid = lax.axis_index("s")
    wid = sid * _SC_CORES + core
    base = wid * chunk
    L = _SC_LANES
    stripe = n_pad // _SC_SUBCORES

    pltpu.sync_copy(ms_hbm.at[pl.ds(base, chunk)], ms_v)
    pltpu.sync_copy(ji_hbm.at[pl.ds(base, chunk)], ji_v)

    ninf = jnp.full((L,), _NEG_INF, jnp.float32)
    big = jnp.full((L,), _BIG_I32, jnp.int32)
    for g in range(n_pad // L):
        bestv[pl.ds(g * L, L)] = ninf
        besti[pl.ds(g * L, L)] = big

    lane_iota = lax.iota(jnp.int32, L)
    for r in range(chunk):
        g, lane = divmod(r, L)
        jv = ji_v[pl.ds(g * L, L)]
        jv = jnp.minimum(jnp.maximum(jv, 0), np.int32(n_pad - 1))
        mv = ms_v[pl.ds(g * L, L)]
        iv = lane_iota + (base + g * L)
        gv = plsc.load_gather(bestv, [jv])
        gi = plsc.load_gather(besti, [jv])
        win = (mv > gv) | ((mv == gv) & (iv < gi))
        msk = (lane_iota == lane) & win & (iv < n_valid)
        plsc.store_scatter(bestv, [jv], mv, mask=msk)
        plsc.store_scatter(besti, [jv], iv, mask=msk)

    pltpu.sync_copy(bestv, shv.at[pl.ds(sid * n_pad, n_pad)])
    pltpu.sync_copy(besti, shi.at[pl.ds(sid * n_pad, n_pad)])
    plsc.subcore_barrier()
    pltpu.sync_copy(shv.at[pl.ds(sid * stripe, stripe)], tv)
    pltpu.sync_copy(shi.at[pl.ds(sid * stripe, stripe)], ti)
    for g in range(stripe // L):
        mgv[pl.ds(g * L, L)] = tv[pl.ds(g * L, L)]
        mgi[pl.ds(g * L, L)] = ti[pl.ds(g * L, L)]
    for s2 in range(1, _SC_SUBCORES):
        pltpu.sync_copy(shv.at[pl.ds(s2 * n_pad + sid * stripe, stripe)], tv)
        pltpu.sync_copy(shi.at[pl.ds(s2 * n_pad + sid * stripe, stripe)], ti)
        for g in range(stripe // L):
            av = mgv[pl.ds(g * L, L)]
            ai = mgi[pl.ds(g * L, L)]
            bv = tv[pl.ds(g * L, L)]
            bi = ti[pl.ds(g * L, L)]
            better = (bv > av) | ((bv == av) & (bi < ai))
            mgv[pl.ds(g * L, L)] = jnp.where(better, bv, av)
            mgi[pl.ds(g * L, L)] = jnp.where(better, bi, ai)

    @pl.when(core == 0)
    def _w0():
        pltpu.sync_copy(mgv, cwv0_hbm.at[pl.ds(sid * stripe, stripe)])
        pltpu.sync_copy(mgi, cwi0_hbm.at[pl.ds(sid * stripe, stripe)])

    @pl.when(core == 1)
    def _w1():
        pltpu.sync_copy(mgv, cwv1_hbm.at[pl.ds(sid * stripe, stripe)])
        pltpu.sync_copy(mgi, cwi1_hbm.at[pl.ds(sid * stripe, stripe)])


def _resolve_body(chunk, n_pad,
                  ms_hbm, ji_hbm, cwv0_hbm, cwi0_hbm, cwv1_hbm, cwi1_hbm,
                  out_hbm, ms_v, ji_v, v0, i0, v1, i1, out_v):
    wid = lax.axis_index("s") * _SC_CORES + lax.axis_index("c")
    base = wid * chunk
    L = _SC_LANES
    pltpu.sync_copy(ms_hbm.at[pl.ds(base, chunk)], ms_v)
    pltpu.sync_copy(ji_hbm.at[pl.ds(base, chunk)], ji_v)
    pltpu.sync_copy(cwv0_hbm, v0)
    pltpu.sync_copy(cwi0_hbm, i0)
    pltpu.sync_copy(cwv1_hbm, v1)
    pltpu.sync_copy(cwi1_hbm, i1)
    for k in range(chunk // L):
        off = k * L
        jraw = ji_v[pl.ds(off, L)]
        jv = jnp.minimum(jnp.maximum(jraw, 0), np.int32(n_pad - 1))
        mv = ms_v[pl.ds(off, L)]
        gv0 = plsc.load_gather(v0, [jv])
        gi0 = plsc.load_gather(i0, [jv])
        gv1 = plsc.load_gather(v1, [jv])
        gi1 = plsc.load_gather(i1, [jv])
        first = (gv0 > gv1) | ((gv0 == gv1) & (gi0 < gi1))
        w = jnp.where(first, gi0, gi1)
        iv = lax.iota(jnp.int32, L) + (base + off)
        out_v[pl.ds(off, L)] = jnp.where(
            (mv > _THR) & (w == iv), jraw, jnp.int32(-1))
    pltpu.sync_copy(out_v, out_hbm.at[pl.ds(base, chunk)])


def kernel(detections_t, detections_t1, W1, b1, W2, b2):
    N_t, D = detections_t.shape
    N_t1 = detections_t1.shape[0]
    E = W2.shape[1]

    chunk = -(-N_t // (_SC_WORKERS * _SC_LANES)) * _SC_LANES
    n_pad = chunk * _SC_WORKERS

    BLK = 200
    grid = N_t // BLK
    sim, rmax, rarg = pl.pallas_call(
        _sim_body,
        grid=(grid,),
        in_specs=[
            pl.BlockSpec((N_t, D), lambda i: (0, 0)),
            pl.BlockSpec((N_t1, D), lambda i: (0, 0)),
            pl.BlockSpec((D, E), lambda i: (0, 0)),
            pl.BlockSpec((1, E), lambda i: (0, 0)),
            pl.BlockSpec((E, E), lambda i: (0, 0)),
            pl.BlockSpec((1, E), lambda i: (0, 0)),
        ],
        out_specs=[
            pl.BlockSpec((BLK, N_t1), lambda i: (i, 0)),
            pl.BlockSpec((BLK, 1), lambda i: (i, 0)),
            pl.BlockSpec((BLK, 1), lambda i: (i, 0)),
        ],
        out_shape=(
            jax.ShapeDtypeStruct((N_t, N_t1), jnp.float32),
            jax.ShapeDtypeStruct((n_pad, 1), jnp.float32),
            jax.ShapeDtypeStruct((n_pad, 1), jnp.int32),
        ),
        scratch_shapes=[
            pltpu.VMEM((N_t, E), jnp.float32),
            pltpu.VMEM((N_t1, E), jnp.float32),
        ],
    )(detections_t, detections_t1, W1, b1.reshape(1, E), W2,
      b2.reshape(1, E))

    mesh = plsc.VectorSubcoreMesh(core_axis_name="c", subcore_axis_name="s",
                                  num_cores=_SC_CORES,
                                  num_subcores=_SC_SUBCORES)
    stripe = n_pad // _SC_SUBCORES
    scatter = functools.partial(
        pl.kernel,
        out_type=(
            jax.ShapeDtypeStruct((n_pad,), jnp.float32),
            jax.ShapeDtypeStruct((n_pad,), jnp.int32),
            jax.ShapeDtypeStruct((n_pad,), jnp.float32),
            jax.ShapeDtypeStruct((n_pad,), jnp.int32),
        ),
        mesh=mesh,
        compiler_params=pltpu.CompilerParams(needs_layout_passes=False),
        scratch_types=[
            pltpu.VMEM((chunk,), jnp.float32),
            pltpu.VMEM((chunk,), jnp.int32),
            pltpu.VMEM((n_pad,), jnp.float32),
            pltpu.VMEM((n_pad,), jnp.int32),
            pltpu.VMEM_SHARED((_SC_SUBCORES * n_pad,), jnp.float32),
            pltpu.VMEM_SHARED((_SC_SUBCORES * n_pad,), jnp.int32),
            pltpu.VMEM((stripe,), jnp.float32),
            pltpu.VMEM((stripe,), jnp.int32),
            pltpu.VMEM((stripe,), jnp.float32),
            pltpu.VMEM((stripe,), jnp.int32),
        ],
    )(functools.partial(_scatter_body, chunk, n_pad, N_t))
    ms_flat = rmax.reshape(n_pad)
    ji_flat = rarg.reshape(n_pad)
    cwv0, cwi0, cwv1, cwi1 = scatter(ms_flat, ji_flat)

    resolve = functools.partial(
        pl.kernel,
        out_type=jax.ShapeDtypeStruct((n_pad,), jnp.int32),
        mesh=mesh,
        compiler_params=pltpu.CompilerParams(needs_layout_passes=False),
        scratch_types=[
            pltpu.VMEM((chunk,), jnp.float32),
            pltpu.VMEM((chunk,), jnp.int32),
            pltpu.VMEM((n_pad,), jnp.float32),
            pltpu.VMEM((n_pad,), jnp.int32),
            pltpu.VMEM((n_pad,), jnp.float32),
            pltpu.VMEM((n_pad,), jnp.int32),
            pltpu.VMEM((chunk,), jnp.int32),
        ],
    )(functools.partial(_resolve_body, chunk, n_pad))
    matches = resolve(ms_flat, ji_flat, cwv0, cwi0, cwv1, cwi1)[:N_t]
    return matches, sim
